# baseline (speedup 1.0000x reference)
import functools

import jax
import jax.numpy as jnp
from jax import lax
from jax.experimental import pallas as pl
from jax.experimental.pallas import tpu as pltpu
from jax.experimental.pallas import tpu_sc as plsc

_N = 400000
_N_FULL = 800000
_C = 128
_NSUB = 32
_W = 128
_NW = 100
_CHUNK = _NW * _W
_NPAD = _NSUB * _CHUNK
_PAD = _NPAD - _N
_NROWS = _NPAD // _W
_ZR = 448
_D = 4


def _sc_body(data_hbm, idx_hbm, out_hbm, idx_v, data_v, zero_v, lsem, ssem,
             zsem):
    wid = lax.axis_index("c") * 16 + lax.axis_index("s")
    row0 = wid * _NW

    zvec = jnp.zeros((16,), jnp.float32)

    @pl.loop(0, _ZR)
    def _(r):
        for c in range(_C // 16):
            zero_v[r, pl.ds(c * 16, 16)] = zvec

    pltpu.sync_copy(idx_hbm.at[row0], idx_v.at[0])
    s0 = idx_v[0, pl.ds(0, 16)][0]
    rn = jnp.minimum(row0 + _NW, _NROWS - 1)
    pltpu.sync_copy(idx_hbm.at[rn], idx_v.at[0])
    s1 = idx_v[0, pl.ds(0, 16)][0]
    zs = jnp.where(wid == 0, 0, s0)
    ze = jnp.where(wid == _NSUB - 1, _N_FULL, s1)

    nblk = (ze - zs) // _ZR

    @pl.loop(0, nblk)
    def _(t):
        pltpu.async_copy(zero_v, out_hbm.at[pl.ds(zs + t * _ZR, _ZR)], zsem)

    pltpu.async_copy(zero_v, out_hbm.at[pl.ds(ze - _ZR, _ZR)], zsem)

    for b in range(2):
        pltpu.async_copy(idx_hbm.at[row0 + b], idx_v.at[b], lsem.at[b])
        pltpu.async_copy(
            data_hbm.at[pl.ds((row0 + b) * _W, _W)], data_v.at[b],
            lsem.at[b])

    @pl.loop(0, nblk + 1)
    def _(t):
        pltpu.make_async_copy(zero_v, out_hbm.at[pl.ds(zs, _ZR)],
                              zsem).wait()

    @pl.loop(0, _NW, step=_D)
    def _(w):
        for b in range(_D):
            ww = w + b
            b2 = (b + 2) % _D
            pltpu.make_async_copy(idx_hbm.at[row0], idx_v.at[b],
                                  lsem.at[b]).wait()
            pltpu.make_async_copy(data_hbm.at[pl.ds(0, _W)], data_v.at[b],
                                  lsem.at[b]).wait()
            pltpu.async_copy(data_v.at[b], out_hbm.at[idx_v.at[b]],
                             ssem.at[b])

            @pl.when(ww >= 2)
            def _():
                pltpu.make_async_copy(data_v.at[b2],
                                      out_hbm.at[idx_v.at[b2]],
                                      ssem.at[b2]).wait()

            @pl.when(ww + 2 < _NW)
            def _():
                r = row0 + ww + 2
                pltpu.async_copy(idx_hbm.at[r], idx_v.at[b2], lsem.at[b2])
                pltpu.async_copy(
                    data_hbm.at[pl.ds(r * _W, _W)], data_v.at[b2],
                    lsem.at[b2])

    for ww in (_NW - 2, _NW - 1):
        b = ww % _D
        pltpu.make_async_copy(data_v.at[b], out_hbm.at[idx_v.at[b]],
                              ssem.at[b]).wait()


@jax.jit
def _octree_pad(data_pad, idx2d):
    mesh = plsc.VectorSubcoreMesh(core_axis_name="c", subcore_axis_name="s")
    run = pl.kernel(
        _sc_body,
        out_type=jax.ShapeDtypeStruct((_N_FULL, _C), jnp.float32),
        mesh=mesh,
        scratch_types=[
            pltpu.VMEM((_D, _W), jnp.int32),
            pltpu.VMEM((_D, _W, _C), jnp.float32),
            pltpu.VMEM((_ZR, _C), jnp.float32),
            pltpu.SemaphoreType.DMA((_D,)),
            pltpu.SemaphoreType.DMA((_D,)),
            pltpu.SemaphoreType.DMA,
        ],
        compiler_params=pltpu.CompilerParams(use_tc_tiling_on_sc=False),
    )
    return run(data_pad, idx2d)


def kernel(data_in, octree):
    idx = octree.astype(jnp.int32)
    idx_pad = jnp.concatenate([idx, idx[_N - _PAD:]])
    data_pad = jnp.concatenate([data_in, data_in[_N - _PAD:]])
    return _octree_pad(data_pad, idx_pad.reshape(_NROWS, _W))


# speedup vs baseline: 1.4212x; 1.4212x over previous
import functools

import jax
import jax.numpy as jnp
from jax import lax
from jax.experimental import pallas as pl
from jax.experimental.pallas import tpu as pltpu
from jax.experimental.pallas import tpu_sc as plsc

_N = 400000
_N_FULL = 800000
_C = 128
_NSUB = 32
_W = 128
_NW = 100
_CHUNK = _NW * _W
_NPAD = _NSUB * _CHUNK
_PAD = _NPAD - _N
_NROWS = _NPAD // _W
_ZR = 448
_D = 4
_RREAL = _N // _W


def _src_base(r):
    return r * _W - jnp.where(r >= _RREAL, _PAD, 0)


def _sc_body(data_hbm, idx_hbm, out_hbm, idx_v, data_v, zero_v, lsem, ssem,
             zsem):
    wid = lax.axis_index("c") * 16 + lax.axis_index("s")
    row0 = wid * _NW

    zvec = jnp.zeros((16,), jnp.float32)

    @pl.loop(0, _ZR)
    def _(r):
        for c in range(_C // 16):
            zero_v[r, pl.ds(c * 16, 16)] = zvec

    pltpu.sync_copy(idx_hbm.at[row0], idx_v.at[0])
    s0 = idx_v[0, pl.ds(0, 16)][0]
    rn = jnp.minimum(row0 + _NW, _NROWS - 1)
    pltpu.sync_copy(idx_hbm.at[rn], idx_v.at[0])
    s1 = idx_v[0, pl.ds(0, 16)][0]
    zs = jnp.where(wid == 0, 0, s0)
    ze = jnp.where(wid == _NSUB - 1, _N_FULL, s1)

    nblk = (ze - zs) // _ZR

    @pl.loop(0, nblk)
    def _(t):
        pltpu.async_copy(zero_v, out_hbm.at[pl.ds(zs + t * _ZR, _ZR)], zsem)

    pltpu.async_copy(zero_v, out_hbm.at[pl.ds(ze - _ZR, _ZR)], zsem)

    for b in range(2):
        pltpu.async_copy(idx_hbm.at[row0 + b], idx_v.at[b], lsem.at[b])
        pltpu.async_copy(
            data_hbm.at[pl.ds(_src_base(row0 + b), _W)], data_v.at[b],
            lsem.at[b])

    @pl.loop(0, nblk + 1)
    def _(t):
        pltpu.make_async_copy(zero_v, out_hbm.at[pl.ds(zs, _ZR)],
                              zsem).wait()

    @pl.loop(0, _NW, step=_D)
    def _(w):
        for b in range(_D):
            ww = w + b
            b2 = (b + 2) % _D
            pltpu.make_async_copy(idx_hbm.at[row0], idx_v.at[b],
                                  lsem.at[b]).wait()
            pltpu.make_async_copy(data_hbm.at[pl.ds(0, _W)], data_v.at[b],
                                  lsem.at[b]).wait()
            pltpu.async_copy(data_v.at[b], out_hbm.at[idx_v.at[b]],
                             ssem.at[b])

            @pl.when(ww >= 2)
            def _():
                pltpu.make_async_copy(data_v.at[b2],
                                      out_hbm.at[idx_v.at[b2]],
                                      ssem.at[b2]).wait()

            @pl.when(ww + 2 < _NW)
            def _():
                r = row0 + ww + 2
                pltpu.async_copy(idx_hbm.at[r], idx_v.at[b2], lsem.at[b2])
                pltpu.async_copy(
                    data_hbm.at[pl.ds(_src_base(r), _W)], data_v.at[b2],
                    lsem.at[b2])

    for ww in (_NW - 2, _NW - 1):
        b = ww % _D
        pltpu.make_async_copy(data_v.at[b], out_hbm.at[idx_v.at[b]],
                              ssem.at[b]).wait()


@jax.jit
def _octree_pad(data_pad, idx2d):
    mesh = plsc.VectorSubcoreMesh(core_axis_name="c", subcore_axis_name="s")
    run = pl.kernel(
        _sc_body,
        out_type=jax.ShapeDtypeStruct((_N_FULL, _C), jnp.float32),
        mesh=mesh,
        scratch_types=[
            pltpu.VMEM((_D, _W), jnp.int32),
            pltpu.VMEM((_D, _W, _C), jnp.float32),
            pltpu.VMEM((_ZR, _C), jnp.float32),
            pltpu.SemaphoreType.DMA((_D,)),
            pltpu.SemaphoreType.DMA((_D,)),
            pltpu.SemaphoreType.DMA,
        ],
        compiler_params=pltpu.CompilerParams(use_tc_tiling_on_sc=False),
    )
    return run(data_pad, idx2d)


def kernel(data_in, octree):
    idx = octree.astype(jnp.int32)
    idx_pad = jnp.concatenate([idx, idx[_N - _PAD:]])
    return _octree_pad(data_in, idx_pad.reshape(_NROWS, _W))
